# baseline (speedup 1.0000x reference)
import functools

import jax
import jax.numpy as jnp
from jax import lax
from jax.experimental import pallas as pl
from jax.experimental.pallas import tpu as pltpu
from jax.experimental.pallas import tpu_sc as plsc

BATCH = 16384
NFACT = 64
NC = 2
NS = 16
L = 16
NW = NC * NS
BW = BATCH // NW


def _sc_body(uids, mids, uft, mft, ubt, mbt, gb, out,
             uid_v, mid_v, usoa, msoa, ub_v, mb_v, gb_v, out_v, sem):
    wid = lax.axis_index("s") * NC + lax.axis_index("c")
    base = wid * BW

    pltpu.sync_copy(uids.at[pl.ds(base, BW)], uid_v)
    pltpu.sync_copy(mids.at[pl.ds(base, BW)], mid_v)
    pltpu.sync_copy(gb, gb_v)

    copies = []
    for d in range(NFACT):
        copies.append(pltpu.async_copy(uft.at[d].at[uid_v], usoa.at[d], sem))
        copies.append(pltpu.async_copy(mft.at[d].at[mid_v], msoa.at[d], sem))
    copies.append(pltpu.async_copy(ubt.at[0].at[uid_v], ub_v, sem))
    copies.append(pltpu.async_copy(mbt.at[0].at[mid_v], mb_v, sem))
    for c in copies:
        c.wait()

    gbv = gb_v[...]

    def g_body(g, carry):
        j0 = g * L
        acc = ub_v[pl.ds(j0, L)] + mb_v[pl.ds(j0, L)] + gbv
        for d in range(NFACT):
            acc = acc + usoa[d, pl.ds(j0, L)] * msoa[d, pl.ds(j0, L)]
        pos = acc >= 0.0
        e = jnp.exp(jnp.where(pos, -acc, acc))
        out_v[pl.ds(j0, L)] = jnp.where(pos, 1.0 / (1.0 + e), e / (1.0 + e))
        return carry

    lax.fori_loop(0, BW // L, g_body, 0)
    pltpu.sync_copy(out_v, out.at[pl.ds(base, BW)])


@jax.jit
def _run(uids, mids, uft, mft, ubt, mbt, gb):
    mesh = plsc.VectorSubcoreMesh(core_axis_name="c", subcore_axis_name="s")
    f = functools.partial(
        pl.kernel,
        out_type=jax.ShapeDtypeStruct((BATCH,), jnp.float32),
        mesh=mesh,
        scratch_types=[
            pltpu.VMEM((BW,), jnp.int32),
            pltpu.VMEM((BW,), jnp.int32),
            pltpu.VMEM((NFACT, BW), jnp.float32),
            pltpu.VMEM((NFACT, BW), jnp.float32),
            pltpu.VMEM((BW,), jnp.float32),
            pltpu.VMEM((BW,), jnp.float32),
            pltpu.VMEM((L,), jnp.float32),
            pltpu.VMEM((BW,), jnp.float32),
            pltpu.SemaphoreType.DMA,
        ],
        compiler_params=pltpu.CompilerParams(
            needs_layout_passes=False, use_tc_tiling_on_sc=False),
    )(_sc_body)
    return f(uids, mids, uft, mft, ubt, mbt, gb)


def kernel(user_ids, movie_ids, user_factors, movie_factors,
           user_bias, movie_bias, global_bias):
    uids = user_ids.astype(jnp.int32)
    mids = movie_ids.astype(jnp.int32)
    gb = jnp.broadcast_to(global_bias.astype(jnp.float32), (L,))
    return _run(uids, mids, user_factors.T, movie_factors.T,
                user_bias.T, movie_bias.T, gb)


# speedup vs baseline: 7.6247x; 7.6247x over previous
import functools

import jax
import jax.numpy as jnp
from jax import lax
from jax.experimental import pallas as pl
from jax.experimental.pallas import tpu as pltpu
from jax.experimental.pallas import tpu_sc as plsc

BATCH = 16384
NFACT = 64
NC = 2
NS = 16
L = 16
NW = NC * NS
BW = BATCH // NW
PB = 128
NPASS = BW // PB
UBROWS = (1000000 + 127) // 128
MBROWS = (100000 + 127) // 128


def _sc_body(uids, mids, uf2, mf2, ub2, mb2, gb2, out,
             uid_v, mid_v, iu_v, im_v, ibu_v, ibm_v,
             u2, m2, ubr, mbr, gb_v, out_v, sem):
    wid = lax.axis_index("s") * NC + lax.axis_index("c")
    base = wid * BW

    pltpu.sync_copy(uids.at[pl.ds(base, BW)], uid_v)
    pltpu.sync_copy(mids.at[pl.ds(base, BW)], mid_v)
    pltpu.sync_copy(gb2, gb_v)

    def idx_body(i, carry):
        j0 = i * L
        u = uid_v[pl.ds(j0, L)]
        m = mid_v[pl.ds(j0, L)]
        iu_v[pl.ds(j0, L)] = u >> 1
        im_v[pl.ds(j0, L)] = m >> 1
        ibu_v[pl.ds(j0, L)] = u >> 7
        ibm_v[pl.ds(j0, L)] = m >> 7
        return carry

    lax.fori_loop(0, BW // L, idx_body, 0)

    lane = lax.iota(jnp.int32, L)
    zeros = jnp.zeros((L,), jnp.int32)
    gbv = plsc.load_gather(gb_v, [zeros, lane])

    for p in range(NPASS):
        p0 = p * PB
        c1 = pltpu.async_copy(uf2.at[iu_v.at[pl.ds(p0, PB)]], u2, sem)
        c2 = pltpu.async_copy(mf2.at[im_v.at[pl.ds(p0, PB)]], m2, sem)
        c3 = pltpu.async_copy(ub2.at[ibu_v.at[pl.ds(p0, PB)]], ubr, sem)
        c4 = pltpu.async_copy(mb2.at[ibm_v.at[pl.ds(p0, PB)]], mbr, sem)
        c1.wait()
        c2.wait()
        c3.wait()
        c4.wait()

        def g_body(g, carry):
            j0 = g * L
            jvec = lane + j0
            u = uid_v[pl.ds(p0 + j0, L)]
            m = mid_v[pl.ds(p0 + j0, L)]
            cu = (u & 1) * NFACT
            cm = (m & 1) * NFACT
            acc = (gbv
                   + plsc.load_gather(ubr, [jvec, u & 127])
                   + plsc.load_gather(mbr, [jvec, m & 127]))
            for d in range(NFACT):
                acc = acc + (plsc.load_gather(u2, [jvec, cu + d])
                             * plsc.load_gather(m2, [jvec, cm + d]))
            pos = acc >= 0.0
            e = jnp.exp(jnp.where(pos, -acc, acc))
            out_v[pl.ds(p0 + j0, L)] = jnp.where(
                pos, 1.0 / (1.0 + e), e / (1.0 + e))
            return carry

        lax.fori_loop(0, PB // L, g_body, 0)

    pltpu.sync_copy(out_v, out.at[pl.ds(base, BW)])


@jax.jit
def _run(uids, mids, uf2, mf2, ub2, mb2, gb2):
    mesh = plsc.VectorSubcoreMesh(core_axis_name="c", subcore_axis_name="s")
    f = functools.partial(
        pl.kernel,
        out_type=jax.ShapeDtypeStruct((BATCH,), jnp.float32),
        mesh=mesh,
        scratch_types=[
            pltpu.VMEM((BW,), jnp.int32),
            pltpu.VMEM((BW,), jnp.int32),
            pltpu.VMEM((BW,), jnp.int32),
            pltpu.VMEM((BW,), jnp.int32),
            pltpu.VMEM((BW,), jnp.int32),
            pltpu.VMEM((BW,), jnp.int32),
            pltpu.VMEM((PB, 128), jnp.float32),
            pltpu.VMEM((PB, 128), jnp.float32),
            pltpu.VMEM((PB, 128), jnp.float32),
            pltpu.VMEM((PB, 128), jnp.float32),
            pltpu.VMEM((8, 128), jnp.float32),
            pltpu.VMEM((BW,), jnp.float32),
            pltpu.SemaphoreType.DMA,
        ],
        compiler_params=pltpu.CompilerParams(
            needs_layout_passes=False, use_tc_tiling_on_sc=True),
    )(_sc_body)
    return f(uids, mids, uf2, mf2, ub2, mb2, gb2)


def kernel(user_ids, movie_ids, user_factors, movie_factors,
           user_bias, movie_bias, global_bias):
    uids = user_ids.astype(jnp.int32)
    mids = movie_ids.astype(jnp.int32)
    uf2 = user_factors.reshape(500000, 128)
    mf2 = movie_factors.reshape(50000, 128)
    ub2 = jnp.pad(user_bias.reshape(-1),
                  (0, UBROWS * 128 - 1000000)).reshape(UBROWS, 128)
    mb2 = jnp.pad(movie_bias.reshape(-1),
                  (0, MBROWS * 128 - 100000)).reshape(MBROWS, 128)
    gb2 = jnp.broadcast_to(global_bias.astype(jnp.float32), (8, 128))
    return _run(uids, mids, uf2, mf2, ub2, mb2, gb2)
